# Initial kernel scaffold; baseline (speedup 1.0000x reference)
import jax
import jax.numpy as jnp
from jax.experimental import pallas as pl


def kernel(h, x, src, dst, distances, bpp_bias, msa_bias, chem_bias, relative_offset, chain_break_mask, W_m1, b_m1, W_m2, b_m2, W_h1, b_h1, W_h2, b_h2, W_c1, b_c1, W_c2, b_c2):
    raise NotImplementedError("write your pallas kernel here")


# speedup vs baseline: 3.7919x; 3.7919x over previous
import functools

import jax
import jax.numpy as jnp
from jax import lax
from jax.experimental import pallas as pl
from jax.experimental.pallas import tpu as pltpu
from jax.experimental.pallas import tpu_sc as plsc

NC = 2
NS = 16
NW = NC * NS
LANES = 16
CH = 80


def _silu(v):
    return v * jax.nn.sigmoid(v)


def _pick_block(total, prefs):
    for b in prefs:
        if b <= total and total % b == 0:
            return b
    return total


def _pre_body(h_ref, w1a_ref, w1b_ref, a_ref, b_ref):
    h = h_ref[...]
    a_ref[...] = jnp.dot(h, w1a_ref[...], preferred_element_type=jnp.float32)
    b_ref[...] = jnp.dot(h, w1b_ref[...], preferred_element_type=jnp.float32)


def _tc_pre(h, w1a, w1b):
    n, d = h.shape
    bn = _pick_block(n, (2000, 1000, 500, 400, 200, 100, 50, 40, 20, 10, 8))
    grid = (n // bn,)
    return pl.pallas_call(
        _pre_body,
        grid=grid,
        in_specs=[
            pl.BlockSpec((bn, d), lambda i: (i, 0)),
            pl.BlockSpec((d, d), lambda i: (0, 0)),
            pl.BlockSpec((d, d), lambda i: (0, 0)),
        ],
        out_specs=[
            pl.BlockSpec((bn, d), lambda i: (i, 0)),
            pl.BlockSpec((bn, d), lambda i: (i, 0)),
        ],
        out_shape=[
            jax.ShapeDtypeStruct((n, d), jnp.float32),
            jax.ShapeDtypeStruct((n, d), jnp.float32),
        ],
    )(h, w1a, w1b)


def _sc_gather_body(epw, ch, dv, dx,
                    a_hbm, b_hbm, xf_hbm, src_hbm, dst_hbm,
                    m1_hbm, df_hbm,
                    src_v, dst_v, buf_a, buf_b, xf_v, buf_df):
    c = lax.axis_index("c")
    s = lax.axis_index("s")
    wid = s * NC + c
    ebase = wid * epw
    pltpu.sync_copy(src_hbm.at[pl.ds(ebase, epw)], src_v)
    pltpu.sync_copy(dst_hbm.at[pl.ds(ebase, epw)], dst_v)
    pltpu.sync_copy(xf_hbm, xf_v)

    zvec = jnp.zeros((LANES,), jnp.float32)

    def zrow(r, carry):
        buf_df[r, :] = zvec
        return carry

    lax.fori_loop(0, ch, zrow, 0)

    def chunk(j, carry):
        off = j * ch
        sidx = src_v.at[pl.ds(off, ch)]
        didx = dst_v.at[pl.ds(off, ch)]
        pltpu.sync_copy(a_hbm.at[sidx], buf_a)
        pltpu.sync_copy(b_hbm.at[didx], buf_b)

        for g in range(ch // LANES):
            il = lax.iota(jnp.int32, LANES) + g * LANES
            sv = src_v[pl.ds(off + g * LANES, LANES)] * 4
            dvv = dst_v[pl.ds(off + g * LANES, LANES)] * 4
            for comp in range(3):
                xs = plsc.load_gather(xf_v, [sv + comp])
                xd = plsc.load_gather(xf_v, [dvv + comp])
                cv = jnp.full((LANES,), comp, jnp.int32)
                plsc.store_scatter(buf_df, [il, cv], xs - xd)

        def add_row(i, carry2):
            r = i // dv
            k = (i % dv) * LANES
            buf_a[r, pl.ds(k, LANES)] = (
                buf_a[r, pl.ds(k, LANES)] + buf_b[r, pl.ds(k, LANES)])
            return carry2

        lax.fori_loop(0, ch * dv, add_row, 0)

        pltpu.sync_copy(buf_a, m1_hbm.at[pl.ds(ebase + off, ch)])
        pltpu.sync_copy(buf_df, df_hbm.at[pl.ds(ebase + off, ch)])
        return carry

    lax.fori_loop(0, epw // ch, chunk, 0)


def _sc_gather(a, b, xf, src, dst, dx):
    e = src.shape[0]
    d = a.shape[1]
    n4 = xf.shape[0]
    epw = e // NW
    dv = d // LANES
    mesh = plsc.VectorSubcoreMesh(core_axis_name="c", subcore_axis_name="s",
                                  num_cores=NC, num_subcores=NS)
    body = functools.partial(_sc_gather_body, epw, CH, dv, dx)
    k = pl.kernel(
        body,
        out_type=[
            jax.ShapeDtypeStruct((e, d), jnp.float32),
            jax.ShapeDtypeStruct((e, dx), jnp.float32),
        ],
        mesh=mesh,
        scratch_types=[
            pltpu.VMEM((epw,), jnp.int32),
            pltpu.VMEM((epw,), jnp.int32),
            pltpu.VMEM((CH, d), jnp.float32),
            pltpu.VMEM((CH, d), jnp.float32),
            pltpu.VMEM((n4,), jnp.float32),
            pltpu.VMEM((CH, dx), jnp.float32),
        ],
        compiler_params=pltpu.CompilerParams(needs_layout_passes=False),
    )
    return k(a, b, xf, src, dst)


def _edge_body(m1p_ref, df_ref, s_ref, w1s_ref, bm1_ref, wm2_ref, bm2_ref,
               wc1_ref, bc1_ref, wc2_ref, bc2_ref, mij_ref, wd_ref):
    s = s_ref[...]
    lane = lax.broadcasted_iota(jnp.int32, s.shape, 1)
    s2 = jnp.where(lane == 0, s * s, s)
    m1 = (m1p_ref[...]
          + jnp.dot(s2, w1s_ref[...], preferred_element_type=jnp.float32)
          + bm1_ref[...])
    hid = _silu(m1)
    mij = _silu(jnp.dot(hid, wm2_ref[...], preferred_element_type=jnp.float32)
                + bm2_ref[...])
    mij_ref[...] = mij
    c1 = _silu(jnp.dot(mij, wc1_ref[...], preferred_element_type=jnp.float32)
               + bc1_ref[...])
    coef = jnp.tanh(jnp.sum(c1 * wc2_ref[...], axis=1, keepdims=True)
                    + bc2_ref[...])
    wd = df_ref[...] * coef
    lane_wd = lax.broadcasted_iota(jnp.int32, wd.shape, 1)
    wd_ref[...] = jnp.where(lane_wd == 3, 1.0, wd)


def _tc_edge(m1pre, df, s8, w1s, bm1, wm2, bm2, wc1, bc1, wc2t, bc2):
    e, d = m1pre.shape
    dx = df.shape[1]
    be = _pick_block(e, (2560, 2048, 2000, 1600, 1280, 1000, 800, 640, 512,
                         400, 320, 256, 200, 160, 128, 80, 40, 16, 8))
    grid = (e // be,)
    zero = lambda i: (0, 0)
    return pl.pallas_call(
        _edge_body,
        grid=grid,
        in_specs=[
            pl.BlockSpec((be, d), lambda i: (i, 0)),
            pl.BlockSpec((be, dx), lambda i: (i, 0)),
            pl.BlockSpec((be, 8), lambda i: (i, 0)),
            pl.BlockSpec((8, d), zero),
            pl.BlockSpec((1, d), zero),
            pl.BlockSpec((d, d), zero),
            pl.BlockSpec((1, d), zero),
            pl.BlockSpec((d, d), zero),
            pl.BlockSpec((1, d), zero),
            pl.BlockSpec((1, d), zero),
            pl.BlockSpec((1, 1), zero),
        ],
        out_specs=[
            pl.BlockSpec((be, d), lambda i: (i, 0)),
            pl.BlockSpec((be, dx), lambda i: (i, 0)),
        ],
        out_shape=[
            jax.ShapeDtypeStruct((e, d), jnp.float32),
            jax.ShapeDtypeStruct((e, dx), jnp.float32),
        ],
    )(m1pre, df, s8, w1s, bm1, wm2, bm2, wc1, bc1, wc2t, bc2)


def _sc_scatter_body(n, epw, ch, dv,
                     mij_hbm, srcm_hbm, pm_hbm,
                     idx_v, mbuf, macc):
    c = lax.axis_index("c")
    s = lax.axis_index("s")
    wid = s * NC + c
    nchw = epw // ch

    zvec = jnp.zeros((LANES,), jnp.float32)

    def zrow_m(i, carry):
        mbuf[i // dv, pl.ds((i % dv) * LANES, LANES)] = zvec
        return carry

    lax.fori_loop(0, ch * dv, zrow_m, 0)

    nps = (n // NS) & ~7
    tail = n - NS * nps

    k = 0
    while k * ch < nps:
        rows = min(ch, nps - k * ch)
        base = s * nps + k * ch
        pltpu.sync_copy(mbuf.at[pl.ds(0, rows)], macc.at[pl.ds(base, rows)])
        k += 1
    if tail:
        @pl.when(s == 0)
        def _():
            pltpu.sync_copy(mbuf.at[pl.ds(0, tail)],
                            macc.at[pl.ds(NS * nps, tail)])
    plsc.subcore_barrier()

    ebase = wid * epw
    pltpu.sync_copy(srcm_hbm.at[wid], idx_v)

    def chunk(j, carry):
        e0 = ebase + j * ch
        pltpu.sync_copy(mij_hbm.at[pl.ds(e0, ch)], mbuf)
        pltpu.sync_copy(mbuf, macc.at[idx_v.at[j]], add=True)
        return carry

    lax.fori_loop(0, nchw, chunk, 0)
    plsc.subcore_barrier()

    k = 0
    while k * ch < nps:
        rows = min(ch, nps - k * ch)
        base = s * nps + k * ch
        pltpu.sync_copy(macc.at[pl.ds(base, rows)],
                        pm_hbm.at[c].at[pl.ds(base, rows)])
        k += 1
    if tail:
        @pl.when(s == 0)
        def _():
            pltpu.sync_copy(macc.at[pl.ds(NS * nps, tail)],
                            pm_hbm.at[c].at[pl.ds(NS * nps, tail)])


def _sc_scatter(mij, srcm, n):
    e, d = mij.shape
    epw = e // NW
    dv = d // LANES
    mesh = plsc.VectorSubcoreMesh(core_axis_name="c", subcore_axis_name="s", num_cores=NC, num_subcores=NS)
    body = functools.partial(_sc_scatter_body, n, epw, CH, dv)
    k = pl.kernel(
        body,
        out_type=[
            jax.ShapeDtypeStruct((NC, n, d), jnp.float32),
        ],
        mesh=mesh,
        scratch_types=[
            pltpu.VMEM((epw // CH, CH), jnp.int32),
            pltpu.VMEM((CH, d), jnp.float32),
            pltpu.VMEM_SHARED((n, d), jnp.float32),
        ],
        compiler_params=pltpu.CompilerParams(needs_layout_passes=False),
    )
    return k(mij, srcm)[0]


def _sc_wd_body(n, epw, ch, dx,
                wd_hbm, src_hbm, pd_hbm,
                src_v, wbuf, dacc4):
    c = lax.axis_index("c")
    s = lax.axis_index("s")
    wid = s * NC + c
    ebase = wid * epw

    zvec = jnp.zeros((LANES,), jnp.float32)

    def zrow(i, carry):
        dacc4[pl.ds(i * LANES, LANES)] = zvec
        return carry

    lax.fori_loop(0, (4 * n) // LANES, zrow, 0)
    pltpu.sync_copy(src_hbm.at[pl.ds(ebase, epw)], src_v)

    il0 = lax.iota(jnp.int32, LANES)

    def chunk(j, carry):
        off = j * ch
        pltpu.sync_copy(wd_hbm.at[pl.ds(ebase + off, ch)], wbuf)
        for g in range(ch // LANES):
            il = il0 + g * LANES
            sv = src_v[pl.ds(off + g * LANES, LANES)] * 4
            for comp in range(4):
                cv = jnp.full((LANES,), comp, jnp.int32)
                val = plsc.load_gather(wbuf, [il, cv])
                plsc.addupdate_scatter(dacc4, [sv + cv], val)
        return carry

    lax.fori_loop(0, epw // ch, chunk, 0)
    pltpu.sync_copy(dacc4, pd_hbm.at[wid])


def _sc_wd(wd, src, n):
    e, dx = wd.shape
    epw = e // NW
    mesh = plsc.VectorSubcoreMesh(core_axis_name="c", subcore_axis_name="s",
                                  num_cores=NC, num_subcores=NS)
    body = functools.partial(_sc_wd_body, n, epw, CH, dx)
    k = pl.kernel(
        body,
        out_type=[
            jax.ShapeDtypeStruct((NW, 4 * n), jnp.float32),
        ],
        mesh=mesh,
        scratch_types=[
            pltpu.VMEM((epw,), jnp.int32),
            pltpu.VMEM((CH, dx), jnp.float32),
            pltpu.VMEM((4 * n,), jnp.float32),
        ],
        compiler_params=pltpu.CompilerParams(needs_layout_passes=False),
    )
    return k(wd, src)[0]


def _psum_body(p_ref, o_ref):
    o_ref[...] = jnp.sum(p_ref[...], axis=0)


def _tc_psum(p):
    nw, m = p.shape
    return pl.pallas_call(
        _psum_body,
        grid=(1,),
        in_specs=[pl.BlockSpec((nw, m), lambda i: (0, 0))],
        out_specs=pl.BlockSpec((m,), lambda i: (0,)),
        out_shape=jax.ShapeDtypeStruct((m,), jnp.float32),
    )(p)


def _node_body(h_ref, xp_ref, pm_ref, pd_ref, wh1a_ref, wh1b_ref, bh1_ref,
               wh2_ref, bh2_ref, hout_ref, xout_ref):
    msum = pm_ref[0] + pm_ref[1]
    dsum = pd_ref[...]
    cnt = jnp.maximum(dsum[:, 3:4], 1.0)
    m_i = msum / cnt
    h = h_ref[...]
    t = _silu(jnp.dot(h, wh1a_ref[...], preferred_element_type=jnp.float32)
              + jnp.dot(m_i, wh1b_ref[...], preferred_element_type=jnp.float32)
              + bh1_ref[...])
    hout_ref[...] = h + jnp.dot(t, wh2_ref[...],
                                preferred_element_type=jnp.float32) + bh2_ref[...]
    pad = jnp.zeros((xp_ref.shape[0], xp_ref.shape[1] - 4), jnp.float32)
    xout_ref[...] = xp_ref[...] + jnp.concatenate([dsum / cnt, pad], axis=1)


def _tc_node(h, xp, pm, pd, wh1a, wh1b, bh1, wh2, bh2):
    n, d = h.shape
    dx = xp.shape[1]
    bn = _pick_block(n, (2000, 1000, 500, 400, 200, 100, 50, 40, 20, 10, 8))
    grid = (n // bn,)
    zero = lambda i: (0, 0)
    return pl.pallas_call(
        _node_body,
        grid=grid,
        in_specs=[
            pl.BlockSpec((bn, d), lambda i: (i, 0)),
            pl.BlockSpec((bn, dx), lambda i: (i, 0)),
            pl.BlockSpec((NC, bn, d), lambda i: (0, i, 0)),
            pl.BlockSpec((bn, 4), lambda i: (i, 0)),
            pl.BlockSpec((d, d), zero),
            pl.BlockSpec((d, d), zero),
            pl.BlockSpec((1, d), zero),
            pl.BlockSpec((d, d), zero),
            pl.BlockSpec((1, d), zero),
        ],
        out_specs=[
            pl.BlockSpec((bn, d), lambda i: (i, 0)),
            pl.BlockSpec((bn, dx), lambda i: (i, 0)),
        ],
        out_shape=[
            jax.ShapeDtypeStruct((n, d), jnp.float32),
            jax.ShapeDtypeStruct((n, dx), jnp.float32),
        ],
    )(h, xp, pm, pd, wh1a, wh1b, bh1, wh2, bh2)


def kernel(h, x, src, dst, distances, bpp_bias, msa_bias, chem_bias,
           relative_offset, chain_break_mask, W_m1, b_m1, W_m2, b_m2,
           W_h1, b_h1, W_h2, b_h2, W_c1, b_c1, W_c2, b_c2):
    n, d = h.shape
    e = src.shape[0]
    dx = 16

    xp = jnp.zeros((n, dx), jnp.float32).at[:, :3].set(x)
    xf = jnp.zeros((n, 4), jnp.float32).at[:, :3].set(x).reshape(-1)
    s8 = jnp.concatenate(
        [jnp.stack([distances, bpp_bias, msa_bias, chem_bias,
                    relative_offset, chain_break_mask], axis=-1),
         jnp.zeros((e, 2), jnp.float32)], axis=-1)
    w1a = W_m1[:d]
    w1b = W_m1[d:2 * d]
    w1s = jnp.concatenate([W_m1[2 * d:], jnp.zeros((8 - (W_m1.shape[0] - 2 * d), d),
                                                   jnp.float32)], axis=0)
    bm1 = b_m1.reshape(1, d)
    bm2 = b_m2.reshape(1, d)
    bc1 = b_c1.reshape(1, d)
    wc2t = W_c2.reshape(1, d)
    bc2 = b_c2.reshape(1, 1)
    wh1a = W_h1[:d]
    wh1b = W_h1[d:]
    bh1 = b_h1.reshape(1, d)
    bh2 = b_h2.reshape(1, d)
    srcm = src.reshape(NW, e // (NW * CH), CH)

    a, b = _tc_pre(h, w1a, w1b)
    m1pre, df = _sc_gather(a, b, xf, src, dst, dx)
    mij, wdw = _tc_edge(m1pre, df, s8, w1s, bm1, W_m2, bm2, W_c1, bc1,
                        wc2t, bc2)
    pm = _sc_scatter(mij, srcm, n)
    pd4 = _tc_psum(_sc_wd(wdw, src, n)).reshape(n, 4)
    h_out, x_out_p = _tc_node(h, xp, pm, pd4, wh1a, wh1b, bh1, W_h2, bh2)
    return (h_out, x_out_p[:, :3])


# speedup vs baseline: 5.2412x; 1.3822x over previous
import functools

import jax
import jax.numpy as jnp
from jax import lax
from jax.experimental import pallas as pl
from jax.experimental.pallas import tpu as pltpu
from jax.experimental.pallas import tpu_sc as plsc

NC = 2
NS = 16
NW = NC * NS
LANES = 16
CH = 80


def _silu(v):
    return v * jax.nn.sigmoid(v)


def _pick_block(total, prefs):
    for b in prefs:
        if b <= total and total % b == 0:
            return b
    return total


def _pre_body(h_ref, w1a_ref, w1b_ref, a_ref, b_ref):
    h = h_ref[...]
    a_ref[...] = jnp.dot(h, w1a_ref[...], preferred_element_type=jnp.float32)
    b_ref[...] = jnp.dot(h, w1b_ref[...], preferred_element_type=jnp.float32)


def _tc_pre(h, w1a, w1b):
    n, d = h.shape
    bn = _pick_block(n, (2000, 1000, 500, 400, 200, 100, 50, 40, 20, 10, 8))
    grid = (n // bn,)
    return pl.pallas_call(
        _pre_body,
        grid=grid,
        in_specs=[
            pl.BlockSpec((bn, d), lambda i: (i, 0)),
            pl.BlockSpec((d, d), lambda i: (0, 0)),
            pl.BlockSpec((d, d), lambda i: (0, 0)),
        ],
        out_specs=[
            pl.BlockSpec((bn, d), lambda i: (i, 0)),
            pl.BlockSpec((bn, d), lambda i: (i, 0)),
        ],
        out_shape=[
            jax.ShapeDtypeStruct((n, d), jnp.float32),
            jax.ShapeDtypeStruct((n, d), jnp.float32),
        ],
    )(h, w1a, w1b)


def _sc_gather_body(epw, ch, dv, dx,
                    a_hbm, b_hbm, xf_hbm, src_hbm, dst_hbm,
                    m1_hbm, df_hbm,
                    src_v, dst_v, xf_v, buf_a0, buf_b0, buf_a1, buf_b1,
                    buf_df, sa0, sb0, sa1, sb1):
    c = lax.axis_index("c")
    s = lax.axis_index("s")
    wid = s * NC + c
    ebase = wid * epw
    nch = epw // ch
    pltpu.sync_copy(src_hbm.at[pl.ds(ebase, epw)], src_v)
    pltpu.sync_copy(dst_hbm.at[pl.ds(ebase, epw)], dst_v)
    pltpu.sync_copy(xf_hbm, xf_v)

    zvec = jnp.zeros((LANES,), jnp.float32)

    def zrow(r, carry):
        buf_df[r, :] = zvec
        return carry

    lax.fori_loop(0, ch, zrow, 0)

    def start_g(j, ba, bb, sa, sb):
        off = j * ch
        pltpu.async_copy(a_hbm.at[src_v.at[pl.ds(off, ch)]], ba, sa)
        pltpu.async_copy(b_hbm.at[dst_v.at[pl.ds(off, ch)]], bb, sb)

    def wait_g(ba, bb, sa, sb):
        pltpu.make_async_copy(a_hbm.at[src_v.at[pl.ds(0, ch)]], ba, sa).wait()
        pltpu.make_async_copy(b_hbm.at[dst_v.at[pl.ds(0, ch)]], bb, sb).wait()

    def process(j, ba, bb):
        off = j * ch
        for g in range(ch // LANES):
            il = lax.iota(jnp.int32, LANES) + g * LANES
            sv = src_v[pl.ds(off + g * LANES, LANES)] * 4
            dvv = dst_v[pl.ds(off + g * LANES, LANES)] * 4
            for comp in range(3):
                xs = plsc.load_gather(xf_v, [sv + comp])
                xd = plsc.load_gather(xf_v, [dvv + comp])
                cv = jnp.full((LANES,), comp, jnp.int32)
                plsc.store_scatter(buf_df, [il, cv], xs - xd)

        def add_row(r, carry2):
            for kk in range(dv):
                ba[r, pl.ds(kk * LANES, LANES)] = (
                    ba[r, pl.ds(kk * LANES, LANES)]
                    + bb[r, pl.ds(kk * LANES, LANES)])
            return carry2

        lax.fori_loop(0, ch, add_row, 0)
        pltpu.sync_copy(ba, m1_hbm.at[pl.ds(ebase + off, ch)])
        pltpu.sync_copy(buf_df, df_hbm.at[pl.ds(ebase + off, ch)])

    start_g(0, buf_a0, buf_b0, sa0, sb0)

    def pair(jj, carry):
        j0 = 2 * jj
        start_g(j0 + 1, buf_a1, buf_b1, sa1, sb1)
        wait_g(buf_a0, buf_b0, sa0, sb0)
        process(j0, buf_a0, buf_b0)
        lax.cond(j0 + 2 < nch,
                 lambda: start_g(j0 + 2, buf_a0, buf_b0, sa0, sb0),
                 lambda: None)
        wait_g(buf_a1, buf_b1, sa1, sb1)
        process(j0 + 1, buf_a1, buf_b1)
        return carry

    lax.fori_loop(0, nch // 2, pair, 0)
    if nch % 2:
        wait_g(buf_a0, buf_b0, sa0, sb0)
        process(nch - 1, buf_a0, buf_b0)


def _sc_gather(a, b, xf, src, dst, dx):
    e = src.shape[0]
    d = a.shape[1]
    n4 = xf.shape[0]
    epw = e // NW
    dv = d // LANES
    mesh = plsc.VectorSubcoreMesh(core_axis_name="c", subcore_axis_name="s",
                                  num_cores=NC, num_subcores=NS)
    body = functools.partial(_sc_gather_body, epw, CH, dv, dx)
    k = pl.kernel(
        body,
        out_type=[
            jax.ShapeDtypeStruct((e, d), jnp.float32),
            jax.ShapeDtypeStruct((e, dx), jnp.float32),
        ],
        mesh=mesh,
        scratch_types=[
            pltpu.VMEM((epw,), jnp.int32),
            pltpu.VMEM((epw,), jnp.int32),
            pltpu.VMEM((n4,), jnp.float32),
            pltpu.VMEM((CH, d), jnp.float32),
            pltpu.VMEM((CH, d), jnp.float32),
            pltpu.VMEM((CH, d), jnp.float32),
            pltpu.VMEM((CH, d), jnp.float32),
            pltpu.VMEM((CH, dx), jnp.float32),
            pltpu.SemaphoreType.DMA,
            pltpu.SemaphoreType.DMA,
            pltpu.SemaphoreType.DMA,
            pltpu.SemaphoreType.DMA,
        ],
        compiler_params=pltpu.CompilerParams(needs_layout_passes=False),
    )
    return k(a, b, xf, src, dst)


def _edge_body(m1p_ref, df_ref, s_ref, w1s_ref, bm1_ref, wm2_ref, bm2_ref,
               wc1_ref, bc1_ref, wc2_ref, bc2_ref, mij_ref, wd_ref):
    s = s_ref[...]
    lane = lax.broadcasted_iota(jnp.int32, s.shape, 1)
    s2 = jnp.where(lane == 0, s * s, s)
    m1 = (m1p_ref[...]
          + jnp.dot(s2, w1s_ref[...], preferred_element_type=jnp.float32)
          + bm1_ref[...])
    hid = _silu(m1)
    mij = _silu(jnp.dot(hid, wm2_ref[...], preferred_element_type=jnp.float32)
                + bm2_ref[...])
    mij_ref[...] = mij
    c1 = _silu(jnp.dot(mij, wc1_ref[...], preferred_element_type=jnp.float32)
               + bc1_ref[...])
    coef = jnp.tanh(jnp.sum(c1 * wc2_ref[...], axis=1, keepdims=True)
                    + bc2_ref[...])
    wd = df_ref[...] * coef
    lane_wd = lax.broadcasted_iota(jnp.int32, wd.shape, 1)
    wd_ref[...] = jnp.where(lane_wd == 3, 1.0, wd)


def _tc_edge(m1pre, df, s8, w1s, bm1, wm2, bm2, wc1, bc1, wc2t, bc2):
    e, d = m1pre.shape
    dx = df.shape[1]
    be = _pick_block(e, (2560, 2048, 2000, 1600, 1280, 1000, 800, 640, 512,
                         400, 320, 256, 200, 160, 128, 80, 40, 16, 8))
    grid = (e // be,)
    zero = lambda i: (0, 0)
    return pl.pallas_call(
        _edge_body,
        grid=grid,
        in_specs=[
            pl.BlockSpec((be, d), lambda i: (i, 0)),
            pl.BlockSpec((be, dx), lambda i: (i, 0)),
            pl.BlockSpec((be, 8), lambda i: (i, 0)),
            pl.BlockSpec((8, d), zero),
            pl.BlockSpec((1, d), zero),
            pl.BlockSpec((d, d), zero),
            pl.BlockSpec((1, d), zero),
            pl.BlockSpec((d, d), zero),
            pl.BlockSpec((1, d), zero),
            pl.BlockSpec((1, d), zero),
            pl.BlockSpec((1, 1), zero),
        ],
        out_specs=[
            pl.BlockSpec((be, d), lambda i: (i, 0)),
            pl.BlockSpec((be, dx), lambda i: (i, 0)),
        ],
        out_shape=[
            jax.ShapeDtypeStruct((e, d), jnp.float32),
            jax.ShapeDtypeStruct((e, dx), jnp.float32),
        ],
    )(m1pre, df, s8, w1s, bm1, wm2, bm2, wc1, bc1, wc2t, bc2)


def _sc_scatter_body(n, epw, ch, dv,
                     mij_hbm, srcm_hbm, pm_hbm,
                     idx_v, mbuf, macc):
    c = lax.axis_index("c")
    s = lax.axis_index("s")
    wid = s * NC + c
    nchw = epw // ch

    zvec = jnp.zeros((LANES,), jnp.float32)

    def zrow_m(i, carry):
        mbuf[i // dv, pl.ds((i % dv) * LANES, LANES)] = zvec
        return carry

    lax.fori_loop(0, ch * dv, zrow_m, 0)

    nps = (n // NS) & ~7
    tail = n - NS * nps

    k = 0
    while k * ch < nps:
        rows = min(ch, nps - k * ch)
        base = s * nps + k * ch
        pltpu.sync_copy(mbuf.at[pl.ds(0, rows)], macc.at[pl.ds(base, rows)])
        k += 1
    if tail:
        @pl.when(s == 0)
        def _():
            pltpu.sync_copy(mbuf.at[pl.ds(0, tail)],
                            macc.at[pl.ds(NS * nps, tail)])
    plsc.subcore_barrier()

    ebase = wid * epw
    pltpu.sync_copy(srcm_hbm.at[wid], idx_v)

    def chunk(j, carry):
        e0 = ebase + j * ch
        pltpu.sync_copy(mij_hbm.at[pl.ds(e0, ch)], mbuf)
        pltpu.sync_copy(mbuf, macc.at[idx_v.at[j]], add=True)
        return carry

    lax.fori_loop(0, nchw, chunk, 0)
    plsc.subcore_barrier()

    k = 0
    while k * ch < nps:
        rows = min(ch, nps - k * ch)
        base = s * nps + k * ch
        pltpu.sync_copy(macc.at[pl.ds(base, rows)],
                        pm_hbm.at[c].at[pl.ds(base, rows)])
        k += 1
    if tail:
        @pl.when(s == 0)
        def _():
            pltpu.sync_copy(macc.at[pl.ds(NS * nps, tail)],
                            pm_hbm.at[c].at[pl.ds(NS * nps, tail)])


def _sc_scatter(mij, srcm, n):
    e, d = mij.shape
    epw = e // NW
    dv = d // LANES
    mesh = plsc.VectorSubcoreMesh(core_axis_name="c", subcore_axis_name="s", num_cores=NC, num_subcores=NS)
    body = functools.partial(_sc_scatter_body, n, epw, CH, dv)
    k = pl.kernel(
        body,
        out_type=[
            jax.ShapeDtypeStruct((NC, n, d), jnp.float32),
        ],
        mesh=mesh,
        scratch_types=[
            pltpu.VMEM((epw // CH, CH), jnp.int32),
            pltpu.VMEM((CH, d), jnp.float32),
            pltpu.VMEM_SHARED((n, d), jnp.float32),
        ],
        compiler_params=pltpu.CompilerParams(needs_layout_passes=False),
    )
    return k(mij, srcm)[0]


def _sc_wd_body(n, epw, ch, dx,
                wd_hbm, src_hbm, pd_hbm,
                src_v, wbuf, dacc4):
    c = lax.axis_index("c")
    s = lax.axis_index("s")
    wid = s * NC + c
    ebase = wid * epw

    zvec = jnp.zeros((LANES,), jnp.float32)

    def zrow(i, carry):
        dacc4[pl.ds(i * LANES, LANES)] = zvec
        return carry

    lax.fori_loop(0, (4 * n) // LANES, zrow, 0)
    pltpu.sync_copy(src_hbm.at[pl.ds(ebase, epw)], src_v)

    il0 = lax.iota(jnp.int32, LANES)

    def chunk(j, carry):
        off = j * ch
        pltpu.sync_copy(wd_hbm.at[pl.ds(ebase + off, ch)], wbuf)
        for g in range(ch // LANES):
            il = il0 + g * LANES
            sv = src_v[pl.ds(off + g * LANES, LANES)] * 4
            for comp in range(4):
                cv = jnp.full((LANES,), comp, jnp.int32)
                val = plsc.load_gather(wbuf, [il, cv])
                plsc.addupdate_scatter(dacc4, [sv + cv], val)
        return carry

    lax.fori_loop(0, epw // ch, chunk, 0)
    pltpu.sync_copy(dacc4, pd_hbm.at[wid])


def _sc_wd(wd, src, n):
    e, dx = wd.shape
    epw = e // NW
    mesh = plsc.VectorSubcoreMesh(core_axis_name="c", subcore_axis_name="s",
                                  num_cores=NC, num_subcores=NS)
    body = functools.partial(_sc_wd_body, n, epw, CH, dx)
    k = pl.kernel(
        body,
        out_type=[
            jax.ShapeDtypeStruct((NW, 4 * n), jnp.float32),
        ],
        mesh=mesh,
        scratch_types=[
            pltpu.VMEM((epw,), jnp.int32),
            pltpu.VMEM((CH, dx), jnp.float32),
            pltpu.VMEM((4 * n,), jnp.float32),
        ],
        compiler_params=pltpu.CompilerParams(needs_layout_passes=False),
    )
    return k(wd, src)[0]


def _psum_body(p_ref, o_ref):
    o_ref[...] = jnp.sum(p_ref[...], axis=0)


def _tc_psum(p):
    nw, m = p.shape
    return pl.pallas_call(
        _psum_body,
        grid=(1,),
        in_specs=[pl.BlockSpec((nw, m), lambda i: (0, 0))],
        out_specs=pl.BlockSpec((m,), lambda i: (0,)),
        out_shape=jax.ShapeDtypeStruct((m,), jnp.float32),
    )(p)


def _node_body(h_ref, xp_ref, pm_ref, pd_ref, wh1a_ref, wh1b_ref, bh1_ref,
               wh2_ref, bh2_ref, hout_ref, xout_ref):
    msum = pm_ref[0] + pm_ref[1]
    dsum = pd_ref[...]
    cnt = jnp.maximum(dsum[:, 3:4], 1.0)
    m_i = msum / cnt
    h = h_ref[...]
    t = _silu(jnp.dot(h, wh1a_ref[...], preferred_element_type=jnp.float32)
              + jnp.dot(m_i, wh1b_ref[...], preferred_element_type=jnp.float32)
              + bh1_ref[...])
    hout_ref[...] = h + jnp.dot(t, wh2_ref[...],
                                preferred_element_type=jnp.float32) + bh2_ref[...]
    pad = jnp.zeros((xp_ref.shape[0], xp_ref.shape[1] - 4), jnp.float32)
    xout_ref[...] = xp_ref[...] + jnp.concatenate([dsum / cnt, pad], axis=1)


def _tc_node(h, xp, pm, pd, wh1a, wh1b, bh1, wh2, bh2):
    n, d = h.shape
    dx = xp.shape[1]
    bn = _pick_block(n, (2000, 1000, 500, 400, 200, 100, 50, 40, 20, 10, 8))
    grid = (n // bn,)
    zero = lambda i: (0, 0)
    return pl.pallas_call(
        _node_body,
        grid=grid,
        in_specs=[
            pl.BlockSpec((bn, d), lambda i: (i, 0)),
            pl.BlockSpec((bn, dx), lambda i: (i, 0)),
            pl.BlockSpec((NC, bn, d), lambda i: (0, i, 0)),
            pl.BlockSpec((bn, 4), lambda i: (i, 0)),
            pl.BlockSpec((d, d), zero),
            pl.BlockSpec((d, d), zero),
            pl.BlockSpec((1, d), zero),
            pl.BlockSpec((d, d), zero),
            pl.BlockSpec((1, d), zero),
        ],
        out_specs=[
            pl.BlockSpec((bn, d), lambda i: (i, 0)),
            pl.BlockSpec((bn, dx), lambda i: (i, 0)),
        ],
        out_shape=[
            jax.ShapeDtypeStruct((n, d), jnp.float32),
            jax.ShapeDtypeStruct((n, dx), jnp.float32),
        ],
    )(h, xp, pm, pd, wh1a, wh1b, bh1, wh2, bh2)


def kernel(h, x, src, dst, distances, bpp_bias, msa_bias, chem_bias,
           relative_offset, chain_break_mask, W_m1, b_m1, W_m2, b_m2,
           W_h1, b_h1, W_h2, b_h2, W_c1, b_c1, W_c2, b_c2):
    n, d = h.shape
    e = src.shape[0]
    dx = 16

    xp = jnp.zeros((n, dx), jnp.float32).at[:, :3].set(x)
    xf = jnp.zeros((n, 4), jnp.float32).at[:, :3].set(x).reshape(-1)
    s8 = jnp.concatenate(
        [jnp.stack([distances, bpp_bias, msa_bias, chem_bias,
                    relative_offset, chain_break_mask], axis=-1),
         jnp.zeros((e, 2), jnp.float32)], axis=-1)
    w1a = W_m1[:d]
    w1b = W_m1[d:2 * d]
    w1s = jnp.concatenate([W_m1[2 * d:], jnp.zeros((8 - (W_m1.shape[0] - 2 * d), d),
                                                   jnp.float32)], axis=0)
    bm1 = b_m1.reshape(1, d)
    bm2 = b_m2.reshape(1, d)
    bc1 = b_c1.reshape(1, d)
    wc2t = W_c2.reshape(1, d)
    bc2 = b_c2.reshape(1, 1)
    wh1a = W_h1[:d]
    wh1b = W_h1[d:]
    bh1 = b_h1.reshape(1, d)
    bh2 = b_h2.reshape(1, d)
    srcm = src.reshape(NW, e // (NW * CH), CH)

    a, b = _tc_pre(h, w1a, w1b)
    m1pre, df = _sc_gather(a, b, xf, src, dst, dx)
    mij, wdw = _tc_edge(m1pre, df, s8, w1s, bm1, W_m2, bm2, W_c1, bc1,
                        wc2t, bc2)
    pm = _sc_scatter(mij, srcm, n)
    pd4 = _tc_psum(_sc_wd(wdw, src, n)).reshape(n, 4)
    h_out, x_out_p = _tc_node(h, xp, pm, pd4, wh1a, wh1b, bh1, W_h2, bh2)
    return (h_out, x_out_p[:, :3])


# speedup vs baseline: 5.9693x; 1.1389x over previous
import functools

import jax
import jax.numpy as jnp
from jax import lax
from jax.experimental import pallas as pl
from jax.experimental.pallas import tpu as pltpu
from jax.experimental.pallas import tpu_sc as plsc

NC = 2
NS = 16
NW = NC * NS
LANES = 16
CH = 80


def _silu(v):
    return v * jax.nn.sigmoid(v)


def _pick_block(total, prefs):
    for b in prefs:
        if b <= total and total % b == 0:
            return b
    return total


def _pre_body(h_ref, w1a_ref, w1b_ref, a_ref, b_ref):
    h = h_ref[...]
    a_ref[...] = jnp.dot(h, w1a_ref[...], preferred_element_type=jnp.float32)
    b_ref[...] = jnp.dot(h, w1b_ref[...], preferred_element_type=jnp.float32)


def _tc_pre(h, w1a, w1b):
    n, d = h.shape
    bn = _pick_block(n, (2000, 1000, 500, 400, 200, 100, 50, 40, 20, 10, 8))
    grid = (n // bn,)
    return pl.pallas_call(
        _pre_body,
        grid=grid,
        in_specs=[
            pl.BlockSpec((bn, d), lambda i: (i, 0)),
            pl.BlockSpec((d, d), lambda i: (0, 0)),
            pl.BlockSpec((d, d), lambda i: (0, 0)),
        ],
        out_specs=[
            pl.BlockSpec((bn, d), lambda i: (i, 0)),
            pl.BlockSpec((bn, d), lambda i: (i, 0)),
        ],
        out_shape=[
            jax.ShapeDtypeStruct((n, d), jnp.float32),
            jax.ShapeDtypeStruct((n, d), jnp.float32),
        ],
    )(h, w1a, w1b)


def _sc_gather_body(epw, ch, dv, dx,
                    a_hbm, b_hbm, xf_hbm, src_hbm, dst_hbm,
                    m1_hbm, df_hbm,
                    src_v, dst_v, xf_v, buf_a0, buf_b0, buf_a1, buf_b1,
                    buf_df, sa0, sb0, sa1, sb1):
    c = lax.axis_index("c")
    s = lax.axis_index("s")
    wid = s * NC + c
    ebase = wid * epw
    nch = epw // ch
    pltpu.sync_copy(src_hbm.at[pl.ds(ebase, epw)], src_v)
    pltpu.sync_copy(dst_hbm.at[pl.ds(ebase, epw)], dst_v)
    pltpu.sync_copy(xf_hbm, xf_v)

    zvec = jnp.zeros((LANES,), jnp.float32)

    def zrow(r, carry):
        buf_df[r, :] = zvec
        return carry

    lax.fori_loop(0, ch, zrow, 0)

    def start_g(j, ba, bb, sa, sb):
        off = j * ch
        pltpu.async_copy(a_hbm.at[src_v.at[pl.ds(off, ch)]], ba, sa)
        pltpu.async_copy(b_hbm.at[dst_v.at[pl.ds(off, ch)]], bb, sb)

    def wait_g(ba, bb, sa, sb):
        pltpu.make_async_copy(a_hbm.at[src_v.at[pl.ds(0, ch)]], ba, sa).wait()
        pltpu.make_async_copy(b_hbm.at[dst_v.at[pl.ds(0, ch)]], bb, sb).wait()

    def process(j, ba, bb):
        off = j * ch
        for g in range(ch // LANES):
            il = lax.iota(jnp.int32, LANES) + g * LANES
            sv = src_v[pl.ds(off + g * LANES, LANES)] * 4
            dvv = dst_v[pl.ds(off + g * LANES, LANES)] * 4
            for comp in range(3):
                xs = plsc.load_gather(xf_v, [sv + comp])
                xd = plsc.load_gather(xf_v, [dvv + comp])
                cv = jnp.full((LANES,), comp, jnp.int32)
                plsc.store_scatter(buf_df, [il, cv], xs - xd)

        def add_row(r, carry2):
            for kk in range(dv):
                ba[r, pl.ds(kk * LANES, LANES)] = (
                    ba[r, pl.ds(kk * LANES, LANES)]
                    + bb[r, pl.ds(kk * LANES, LANES)])
            return carry2

        lax.fori_loop(0, ch, add_row, 0)
        pltpu.sync_copy(ba, m1_hbm.at[pl.ds(ebase + off, ch)])
        pltpu.sync_copy(buf_df, df_hbm.at[pl.ds(ebase + off, ch)])

    start_g(0, buf_a0, buf_b0, sa0, sb0)

    def pair(jj, carry):
        j0 = 2 * jj
        start_g(j0 + 1, buf_a1, buf_b1, sa1, sb1)
        wait_g(buf_a0, buf_b0, sa0, sb0)
        process(j0, buf_a0, buf_b0)
        lax.cond(j0 + 2 < nch,
                 lambda: start_g(j0 + 2, buf_a0, buf_b0, sa0, sb0),
                 lambda: None)
        wait_g(buf_a1, buf_b1, sa1, sb1)
        process(j0 + 1, buf_a1, buf_b1)
        return carry

    lax.fori_loop(0, nch // 2, pair, 0)
    if nch % 2:
        wait_g(buf_a0, buf_b0, sa0, sb0)
        process(nch - 1, buf_a0, buf_b0)


def _sc_gather(a, b, xf, src, dst, dx):
    e = src.shape[0]
    d = a.shape[1]
    n4 = xf.shape[0]
    epw = e // NW
    dv = d // LANES
    mesh = plsc.VectorSubcoreMesh(core_axis_name="c", subcore_axis_name="s",
                                  num_cores=NC, num_subcores=NS)
    body = functools.partial(_sc_gather_body, epw, CH, dv, dx)
    k = pl.kernel(
        body,
        out_type=[
            jax.ShapeDtypeStruct((e, d), jnp.float32),
            jax.ShapeDtypeStruct((e, dx), jnp.float32),
        ],
        mesh=mesh,
        scratch_types=[
            pltpu.VMEM((epw,), jnp.int32),
            pltpu.VMEM((epw,), jnp.int32),
            pltpu.VMEM((n4,), jnp.float32),
            pltpu.VMEM((CH, d), jnp.float32),
            pltpu.VMEM((CH, d), jnp.float32),
            pltpu.VMEM((CH, d), jnp.float32),
            pltpu.VMEM((CH, d), jnp.float32),
            pltpu.VMEM((CH, dx), jnp.float32),
            pltpu.SemaphoreType.DMA,
            pltpu.SemaphoreType.DMA,
            pltpu.SemaphoreType.DMA,
            pltpu.SemaphoreType.DMA,
        ],
        compiler_params=pltpu.CompilerParams(needs_layout_passes=False),
    )
    return k(a, b, xf, src, dst)


def _edge_body(m1p_ref, df_ref, s_ref, w1s_ref, bm1_ref, wm2_ref, bm2_ref,
               wc1_ref, bc1_ref, wc2_ref, bc2_ref, mij_ref, wd_ref):
    s = s_ref[...]
    lane = lax.broadcasted_iota(jnp.int32, s.shape, 1)
    s2 = jnp.where(lane == 0, s * s, s)
    m1 = (m1p_ref[...]
          + jnp.dot(s2, w1s_ref[...], preferred_element_type=jnp.float32)
          + bm1_ref[...])
    hid = _silu(m1)
    mij = _silu(jnp.dot(hid, wm2_ref[...], preferred_element_type=jnp.float32)
                + bm2_ref[...])
    mij_ref[...] = mij
    c1 = _silu(jnp.dot(mij, wc1_ref[...], preferred_element_type=jnp.float32)
               + bc1_ref[...])
    coef = jnp.tanh(jnp.sum(c1 * wc2_ref[...], axis=1, keepdims=True)
                    + bc2_ref[...])
    wd = df_ref[...] * coef
    lane_wd = lax.broadcasted_iota(jnp.int32, wd.shape, 1)
    wd_ref[...] = jnp.where(lane_wd == 3, 1.0, wd)


def _tc_edge(m1pre, df, s8, w1s, bm1, wm2, bm2, wc1, bc1, wc2t, bc2):
    e, d = m1pre.shape
    dx = df.shape[1]
    be = _pick_block(e, (2560, 2048, 2000, 1600, 1280, 1000, 800, 640, 512,
                         400, 320, 256, 200, 160, 128, 80, 40, 16, 8))
    grid = (e // be,)
    zero = lambda i: (0, 0)
    return pl.pallas_call(
        _edge_body,
        grid=grid,
        in_specs=[
            pl.BlockSpec((be, d), lambda i: (i, 0)),
            pl.BlockSpec((be, dx), lambda i: (i, 0)),
            pl.BlockSpec((be, 8), lambda i: (i, 0)),
            pl.BlockSpec((8, d), zero),
            pl.BlockSpec((1, d), zero),
            pl.BlockSpec((d, d), zero),
            pl.BlockSpec((1, d), zero),
            pl.BlockSpec((d, d), zero),
            pl.BlockSpec((1, d), zero),
            pl.BlockSpec((1, d), zero),
            pl.BlockSpec((1, 1), zero),
        ],
        out_specs=[
            pl.BlockSpec((be, d), lambda i: (i, 0)),
            pl.BlockSpec((be, dx), lambda i: (i, 0)),
        ],
        out_shape=[
            jax.ShapeDtypeStruct((e, d), jnp.float32),
            jax.ShapeDtypeStruct((e, dx), jnp.float32),
        ],
    )(m1pre, df, s8, w1s, bm1, wm2, bm2, wc1, bc1, wc2t, bc2)


def _sc_scatter_body(n, epw, ch, dv,
                     mij_hbm, srcm_hbm, pm_hbm,
                     idx_v, mbuf, mbuf1, macc, sm0, sm1):
    c = lax.axis_index("c")
    s = lax.axis_index("s")
    wid = s * NC + c
    nchw = epw // ch

    zvec = jnp.zeros((LANES,), jnp.float32)

    def zrow_m(i, carry):
        mbuf[i // dv, pl.ds((i % dv) * LANES, LANES)] = zvec
        return carry

    lax.fori_loop(0, ch * dv, zrow_m, 0)

    nps = (n // NS) & ~7
    tail = n - NS * nps

    k = 0
    while k * ch < nps:
        rows = min(ch, nps - k * ch)
        base = s * nps + k * ch
        pltpu.sync_copy(mbuf.at[pl.ds(0, rows)], macc.at[pl.ds(base, rows)])
        k += 1
    if tail:
        @pl.when(s == 0)
        def _():
            pltpu.sync_copy(mbuf.at[pl.ds(0, tail)],
                            macc.at[pl.ds(NS * nps, tail)])
    plsc.subcore_barrier()

    ebase = wid * epw
    pltpu.sync_copy(srcm_hbm.at[wid], idx_v)

    def start_l(j, buf, sem):
        pltpu.async_copy(mij_hbm.at[pl.ds(ebase + j * ch, ch)], buf, sem)

    def wait_l(buf, sem):
        pltpu.make_async_copy(mij_hbm.at[pl.ds(ebase, ch)], buf, sem).wait()

    start_l(0, mbuf, sm0)

    def pair(jj, carry):
        j0 = 2 * jj
        start_l(j0 + 1, mbuf1, sm1)
        wait_l(mbuf, sm0)
        pltpu.sync_copy(mbuf, macc.at[idx_v.at[j0]], add=True)
        lax.cond(j0 + 2 < nchw,
                 lambda: start_l(j0 + 2, mbuf, sm0),
                 lambda: None)
        wait_l(mbuf1, sm1)
        pltpu.sync_copy(mbuf1, macc.at[idx_v.at[j0 + 1]], add=True)
        return carry

    lax.fori_loop(0, nchw // 2, pair, 0)
    if nchw % 2:
        wait_l(mbuf, sm0)
        pltpu.sync_copy(mbuf, macc.at[idx_v.at[nchw - 1]], add=True)
    plsc.subcore_barrier()

    k = 0
    while k * ch < nps:
        rows = min(ch, nps - k * ch)
        base = s * nps + k * ch
        pltpu.sync_copy(macc.at[pl.ds(base, rows)],
                        pm_hbm.at[c].at[pl.ds(base, rows)])
        k += 1
    if tail:
        @pl.when(s == 0)
        def _():
            pltpu.sync_copy(macc.at[pl.ds(NS * nps, tail)],
                            pm_hbm.at[c].at[pl.ds(NS * nps, tail)])


def _sc_scatter(mij, srcm, n):
    e, d = mij.shape
    epw = e // NW
    dv = d // LANES
    mesh = plsc.VectorSubcoreMesh(core_axis_name="c", subcore_axis_name="s", num_cores=NC, num_subcores=NS)
    body = functools.partial(_sc_scatter_body, n, epw, CH, dv)
    k = pl.kernel(
        body,
        out_type=[
            jax.ShapeDtypeStruct((NC, n, d), jnp.float32),
        ],
        mesh=mesh,
        scratch_types=[
            pltpu.VMEM((epw // CH, CH), jnp.int32),
            pltpu.VMEM((CH, d), jnp.float32),
            pltpu.VMEM((CH, d), jnp.float32),
            pltpu.VMEM_SHARED((n, d), jnp.float32),
            pltpu.SemaphoreType.DMA,
            pltpu.SemaphoreType.DMA,
        ],
        compiler_params=pltpu.CompilerParams(needs_layout_passes=False),
    )
    return k(mij, srcm)[0]


def _sc_wd_body(n, epw, ch, dx,
                wd_hbm, src_hbm, pd_hbm,
                src_v, wbuf, wbuf1, dacc4, sw0, sw1):
    c = lax.axis_index("c")
    s = lax.axis_index("s")
    wid = s * NC + c
    ebase = wid * epw
    nch = epw // ch

    zvec = jnp.zeros((LANES,), jnp.float32)

    def zrow(i, carry):
        dacc4[pl.ds(i * LANES, LANES)] = zvec
        return carry

    lax.fori_loop(0, (4 * n) // LANES, zrow, 0)
    pltpu.sync_copy(src_hbm.at[pl.ds(ebase, epw)], src_v)

    il0 = lax.iota(jnp.int32, LANES)

    def start_l(j, buf, sem):
        pltpu.async_copy(wd_hbm.at[pl.ds(ebase + j * ch, ch)], buf, sem)

    def wait_l(buf, sem):
        pltpu.make_async_copy(wd_hbm.at[pl.ds(ebase, ch)], buf, sem).wait()

    def process(j, buf):
        off = j * ch
        for g in range(ch // LANES):
            il = il0 + g * LANES
            sv = src_v[pl.ds(off + g * LANES, LANES)] * 4
            for comp in range(4):
                cv = jnp.full((LANES,), comp, jnp.int32)
                val = plsc.load_gather(buf, [il, cv])
                plsc.addupdate_scatter(dacc4, [sv + cv], val)

    start_l(0, wbuf, sw0)

    def pair(jj, carry):
        j0 = 2 * jj
        start_l(j0 + 1, wbuf1, sw1)
        wait_l(wbuf, sw0)
        process(j0, wbuf)
        lax.cond(j0 + 2 < nch,
                 lambda: start_l(j0 + 2, wbuf, sw0),
                 lambda: None)
        wait_l(wbuf1, sw1)
        process(j0 + 1, wbuf1)
        return carry

    lax.fori_loop(0, nch // 2, pair, 0)
    if nch % 2:
        wait_l(wbuf, sw0)
        process(nch - 1, wbuf)
    pltpu.sync_copy(dacc4, pd_hbm.at[wid])


def _sc_wd(wd, src, n):
    e, dx = wd.shape
    epw = e // NW
    mesh = plsc.VectorSubcoreMesh(core_axis_name="c", subcore_axis_name="s",
                                  num_cores=NC, num_subcores=NS)
    body = functools.partial(_sc_wd_body, n, epw, CH, dx)
    k = pl.kernel(
        body,
        out_type=[
            jax.ShapeDtypeStruct((NW, 4 * n), jnp.float32),
        ],
        mesh=mesh,
        scratch_types=[
            pltpu.VMEM((epw,), jnp.int32),
            pltpu.VMEM((CH, dx), jnp.float32),
            pltpu.VMEM((CH, dx), jnp.float32),
            pltpu.VMEM((4 * n,), jnp.float32),
            pltpu.SemaphoreType.DMA,
            pltpu.SemaphoreType.DMA,
        ],
        compiler_params=pltpu.CompilerParams(needs_layout_passes=False),
    )
    return k(wd, src)[0]


def _psum_body(p_ref, o_ref):
    o_ref[...] = jnp.sum(p_ref[...], axis=0)


def _tc_psum(p):
    nw, m = p.shape
    return pl.pallas_call(
        _psum_body,
        grid=(1,),
        in_specs=[pl.BlockSpec((nw, m), lambda i: (0, 0))],
        out_specs=pl.BlockSpec((m,), lambda i: (0,)),
        out_shape=jax.ShapeDtypeStruct((m,), jnp.float32),
    )(p)


def _node_body(h_ref, xp_ref, pm_ref, pd_ref, wh1a_ref, wh1b_ref, bh1_ref,
               wh2_ref, bh2_ref, hout_ref, xout_ref):
    msum = pm_ref[0] + pm_ref[1]
    dsum = pd_ref[...]
    cnt = jnp.maximum(dsum[:, 3:4], 1.0)
    m_i = msum / cnt
    h = h_ref[...]
    t = _silu(jnp.dot(h, wh1a_ref[...], preferred_element_type=jnp.float32)
              + jnp.dot(m_i, wh1b_ref[...], preferred_element_type=jnp.float32)
              + bh1_ref[...])
    hout_ref[...] = h + jnp.dot(t, wh2_ref[...],
                                preferred_element_type=jnp.float32) + bh2_ref[...]
    pad = jnp.zeros((xp_ref.shape[0], xp_ref.shape[1] - 4), jnp.float32)
    xout_ref[...] = xp_ref[...] + jnp.concatenate([dsum / cnt, pad], axis=1)


def _tc_node(h, xp, pm, pd, wh1a, wh1b, bh1, wh2, bh2):
    n, d = h.shape
    dx = xp.shape[1]
    bn = _pick_block(n, (2000, 1000, 500, 400, 200, 100, 50, 40, 20, 10, 8))
    grid = (n // bn,)
    zero = lambda i: (0, 0)
    return pl.pallas_call(
        _node_body,
        grid=grid,
        in_specs=[
            pl.BlockSpec((bn, d), lambda i: (i, 0)),
            pl.BlockSpec((bn, dx), lambda i: (i, 0)),
            pl.BlockSpec((NC, bn, d), lambda i: (0, i, 0)),
            pl.BlockSpec((bn, 4), lambda i: (i, 0)),
            pl.BlockSpec((d, d), zero),
            pl.BlockSpec((d, d), zero),
            pl.BlockSpec((1, d), zero),
            pl.BlockSpec((d, d), zero),
            pl.BlockSpec((1, d), zero),
        ],
        out_specs=[
            pl.BlockSpec((bn, d), lambda i: (i, 0)),
            pl.BlockSpec((bn, dx), lambda i: (i, 0)),
        ],
        out_shape=[
            jax.ShapeDtypeStruct((n, d), jnp.float32),
            jax.ShapeDtypeStruct((n, dx), jnp.float32),
        ],
    )(h, xp, pm, pd, wh1a, wh1b, bh1, wh2, bh2)


def kernel(h, x, src, dst, distances, bpp_bias, msa_bias, chem_bias,
           relative_offset, chain_break_mask, W_m1, b_m1, W_m2, b_m2,
           W_h1, b_h1, W_h2, b_h2, W_c1, b_c1, W_c2, b_c2):
    n, d = h.shape
    e = src.shape[0]
    dx = 16

    xp = jnp.zeros((n, dx), jnp.float32).at[:, :3].set(x)
    xf = jnp.zeros((n, 4), jnp.float32).at[:, :3].set(x).reshape(-1)
    s8 = jnp.concatenate(
        [jnp.stack([distances, bpp_bias, msa_bias, chem_bias,
                    relative_offset, chain_break_mask], axis=-1),
         jnp.zeros((e, 2), jnp.float32)], axis=-1)
    w1a = W_m1[:d]
    w1b = W_m1[d:2 * d]
    w1s = jnp.concatenate([W_m1[2 * d:], jnp.zeros((8 - (W_m1.shape[0] - 2 * d), d),
                                                   jnp.float32)], axis=0)
    bm1 = b_m1.reshape(1, d)
    bm2 = b_m2.reshape(1, d)
    bc1 = b_c1.reshape(1, d)
    wc2t = W_c2.reshape(1, d)
    bc2 = b_c2.reshape(1, 1)
    wh1a = W_h1[:d]
    wh1b = W_h1[d:]
    bh1 = b_h1.reshape(1, d)
    bh2 = b_h2.reshape(1, d)
    srcm = src.reshape(NW, e // (NW * CH), CH)

    a, b = _tc_pre(h, w1a, w1b)
    m1pre, df = _sc_gather(a, b, xf, src, dst, dx)
    mij, wdw = _tc_edge(m1pre, df, s8, w1s, bm1, W_m2, bm2, W_c1, bc1,
                        wc2t, bc2)
    pm = _sc_scatter(mij, srcm, n)
    pd4 = _tc_psum(_sc_wd(wdw, src, n)).reshape(n, 4)
    h_out, x_out_p = _tc_node(h, xp, pm, pd4, wh1a, wh1b, bh1, W_h2, bh2)
    return (h_out, x_out_p[:, :3])


# speedup vs baseline: 7.0087x; 1.1741x over previous
import functools

import jax
import jax.numpy as jnp
from jax import lax
from jax.experimental import pallas as pl
from jax.experimental.pallas import tpu as pltpu
from jax.experimental.pallas import tpu_sc as plsc

NC = 2
NS = 16
NW = NC * NS
LANES = 16
CH = 80


def _silu(v):
    return v * jax.nn.sigmoid(v)


def _pick_block(total, prefs):
    for b in prefs:
        if b <= total and total % b == 0:
            return b
    return total


def _pre_body(h_ref, w1a_ref, w1b_ref, a_ref, b_ref):
    h = h_ref[...]
    a_ref[...] = jnp.dot(h, w1a_ref[...], preferred_element_type=jnp.float32)
    b_ref[...] = jnp.dot(h, w1b_ref[...], preferred_element_type=jnp.float32)


def _tc_pre(h, w1a, w1b):
    n, d = h.shape
    bn = _pick_block(n, (2000, 1000, 500, 400, 200, 100, 50, 40, 20, 10, 8))
    grid = (n // bn,)
    return pl.pallas_call(
        _pre_body,
        grid=grid,
        in_specs=[
            pl.BlockSpec((bn, d), lambda i: (i, 0)),
            pl.BlockSpec((d, d), lambda i: (0, 0)),
            pl.BlockSpec((d, d), lambda i: (0, 0)),
        ],
        out_specs=[
            pl.BlockSpec((bn, d), lambda i: (i, 0)),
            pl.BlockSpec((bn, d), lambda i: (i, 0)),
        ],
        out_shape=[
            jax.ShapeDtypeStruct((n, d), jnp.float32),
            jax.ShapeDtypeStruct((n, d), jnp.float32),
        ],
    )(h, w1a, w1b)


def _sc_gather_body(epw, ch, dv, dx,
                    a_hbm, b_hbm, xf_hbm, src_hbm, dst_hbm,
                    m1_hbm, df_hbm,
                    src_v, dst_v, xf_v, buf_a0, buf_b0, buf_a1, buf_b1,
                    buf_df, sa0, sb0, sa1, sb1):
    c = lax.axis_index("c")
    s = lax.axis_index("s")
    wid = s * NC + c
    ebase = wid * epw
    nch = epw // ch
    pltpu.sync_copy(src_hbm.at[pl.ds(ebase, epw)], src_v)
    pltpu.sync_copy(dst_hbm.at[pl.ds(ebase, epw)], dst_v)
    pltpu.sync_copy(xf_hbm, xf_v)

    zvec = jnp.zeros((LANES,), jnp.float32)

    def zrow(r, carry):
        buf_df[r, :] = zvec
        return carry

    lax.fori_loop(0, ch, zrow, 0)

    def start_g(j, ba, bb, sa, sb):
        off = j * ch
        pltpu.async_copy(a_hbm.at[src_v.at[pl.ds(off, ch)]], ba, sa)
        pltpu.async_copy(b_hbm.at[dst_v.at[pl.ds(off, ch)]], bb, sb)

    def wait_g(ba, bb, sa, sb):
        pltpu.make_async_copy(a_hbm.at[src_v.at[pl.ds(0, ch)]], ba, sa).wait()
        pltpu.make_async_copy(b_hbm.at[dst_v.at[pl.ds(0, ch)]], bb, sb).wait()

    def process(j, ba, bb):
        off = j * ch
        for g in range(ch // LANES):
            il = lax.iota(jnp.int32, LANES) + g * LANES
            sv = src_v[pl.ds(off + g * LANES, LANES)] * 4
            dvv = dst_v[pl.ds(off + g * LANES, LANES)] * 4
            for comp in range(3):
                xs = plsc.load_gather(xf_v, [sv + comp])
                xd = plsc.load_gather(xf_v, [dvv + comp])
                cv = jnp.full((LANES,), comp, jnp.int32)
                plsc.store_scatter(buf_df, [il, cv], xs - xd)

        def add_row(r, carry2):
            for kk in range(dv):
                ba[r, pl.ds(kk * LANES, LANES)] = (
                    ba[r, pl.ds(kk * LANES, LANES)]
                    + bb[r, pl.ds(kk * LANES, LANES)])
            return carry2

        lax.fori_loop(0, ch, add_row, 0)
        pltpu.sync_copy(ba, m1_hbm.at[pl.ds(ebase + off, ch)])
        pltpu.sync_copy(buf_df, df_hbm.at[pl.ds(ebase + off, ch)])

    start_g(0, buf_a0, buf_b0, sa0, sb0)

    def pair(jj, carry):
        j0 = 2 * jj
        start_g(j0 + 1, buf_a1, buf_b1, sa1, sb1)
        wait_g(buf_a0, buf_b0, sa0, sb0)
        process(j0, buf_a0, buf_b0)
        lax.cond(j0 + 2 < nch,
                 lambda: start_g(j0 + 2, buf_a0, buf_b0, sa0, sb0),
                 lambda: None)
        wait_g(buf_a1, buf_b1, sa1, sb1)
        process(j0 + 1, buf_a1, buf_b1)
        return carry

    lax.fori_loop(0, nch // 2, pair, 0)
    if nch % 2:
        wait_g(buf_a0, buf_b0, sa0, sb0)
        process(nch - 1, buf_a0, buf_b0)


def _sc_gather(a, b, xf, src, dst, dx):
    e = src.shape[0]
    d = a.shape[1]
    n4 = xf.shape[0]
    epw = e // NW
    dv = d // LANES
    mesh = plsc.VectorSubcoreMesh(core_axis_name="c", subcore_axis_name="s",
                                  num_cores=NC, num_subcores=NS)
    body = functools.partial(_sc_gather_body, epw, CH, dv, dx)
    k = pl.kernel(
        body,
        out_type=[
            jax.ShapeDtypeStruct((e, d), jnp.float32),
            jax.ShapeDtypeStruct((e, dx), jnp.float32),
        ],
        mesh=mesh,
        scratch_types=[
            pltpu.VMEM((epw,), jnp.int32),
            pltpu.VMEM((epw,), jnp.int32),
            pltpu.VMEM((n4,), jnp.float32),
            pltpu.VMEM((CH, d), jnp.float32),
            pltpu.VMEM((CH, d), jnp.float32),
            pltpu.VMEM((CH, d), jnp.float32),
            pltpu.VMEM((CH, d), jnp.float32),
            pltpu.VMEM((CH, dx), jnp.float32),
            pltpu.SemaphoreType.DMA,
            pltpu.SemaphoreType.DMA,
            pltpu.SemaphoreType.DMA,
            pltpu.SemaphoreType.DMA,
        ],
        compiler_params=pltpu.CompilerParams(needs_layout_passes=False),
    )
    return k(a, b, xf, src, dst)


def _edge_body(m1p_ref, df_ref, s_ref, w1s_ref, bm1_ref, wm2_ref, bm2_ref,
               wc1_ref, bc1_ref, wc2_ref, bc2_ref, mij_ref, wd_ref):
    s = s_ref[...]
    lane = lax.broadcasted_iota(jnp.int32, s.shape, 1)
    s2 = jnp.where(lane == 0, s * s, s)
    m1 = (m1p_ref[...]
          + jnp.dot(s2, w1s_ref[...], preferred_element_type=jnp.float32)
          + bm1_ref[...])
    hid = _silu(m1)
    mij = _silu(jnp.dot(hid.astype(jnp.bfloat16), wm2_ref[...],
                        preferred_element_type=jnp.float32)
                + bm2_ref[...])
    mij_ref[...] = mij
    c1 = _silu(jnp.dot(mij.astype(jnp.bfloat16), wc1_ref[...],
               preferred_element_type=jnp.float32)
               + bc1_ref[...])
    coef = jnp.tanh(jnp.sum(c1 * wc2_ref[...], axis=1, keepdims=True)
                    + bc2_ref[...])
    wd = df_ref[...] * coef
    lane_wd = lax.broadcasted_iota(jnp.int32, wd.shape, 1)
    wd_ref[...] = jnp.where(lane_wd == 3, 1.0, wd)


def _tc_edge(m1pre, df, s8, w1s, bm1, wm2, bm2, wc1, bc1, wc2t, bc2):
    e, d = m1pre.shape
    dx = df.shape[1]
    be = _pick_block(e, (2560, 2048, 2000, 1600, 1280, 1000, 800, 640, 512,
                         400, 320, 256, 200, 160, 128, 80, 40, 16, 8))
    grid = (e // be,)
    zero = lambda i: (0, 0)
    return pl.pallas_call(
        _edge_body,
        grid=grid,
        in_specs=[
            pl.BlockSpec((be, d), lambda i: (i, 0)),
            pl.BlockSpec((be, dx), lambda i: (i, 0)),
            pl.BlockSpec((be, 8), lambda i: (i, 0)),
            pl.BlockSpec((8, d), zero),
            pl.BlockSpec((1, d), zero),
            pl.BlockSpec((d, d), zero),
            pl.BlockSpec((1, d), zero),
            pl.BlockSpec((d, d), zero),
            pl.BlockSpec((1, d), zero),
            pl.BlockSpec((1, d), zero),
            pl.BlockSpec((1, 1), zero),
        ],
        out_specs=[
            pl.BlockSpec((be, d), lambda i: (i, 0)),
            pl.BlockSpec((be, dx), lambda i: (i, 0)),
        ],
        out_shape=[
            jax.ShapeDtypeStruct((e, d), jnp.float32),
            jax.ShapeDtypeStruct((e, dx), jnp.float32),
        ],
    )(m1pre, df, s8, w1s, bm1, wm2, bm2, wc1, bc1, wc2t, bc2)


def _sc_scatter_body(n, epw, ch, dv,
                     mij_hbm, srcm_hbm, pm_hbm,
                     idx_v, mbuf, mbuf1, macc, sm0, sm1):
    c = lax.axis_index("c")
    s = lax.axis_index("s")
    wid = s * NC + c
    nchw = epw // ch

    zvec = jnp.zeros((LANES,), jnp.float32)

    def zrow_m(i, carry):
        mbuf[i // dv, pl.ds((i % dv) * LANES, LANES)] = zvec
        return carry

    lax.fori_loop(0, ch * dv, zrow_m, 0)

    nps = (n // NS) & ~7
    tail = n - NS * nps

    k = 0
    while k * ch < nps:
        rows = min(ch, nps - k * ch)
        base = s * nps + k * ch
        pltpu.sync_copy(mbuf.at[pl.ds(0, rows)], macc.at[pl.ds(base, rows)])
        k += 1
    if tail:
        @pl.when(s == 0)
        def _():
            pltpu.sync_copy(mbuf.at[pl.ds(0, tail)],
                            macc.at[pl.ds(NS * nps, tail)])
    plsc.subcore_barrier()

    ebase = wid * epw
    pltpu.sync_copy(srcm_hbm.at[wid], idx_v)

    def start_l(j, buf, sem):
        pltpu.async_copy(mij_hbm.at[pl.ds(ebase + j * ch, ch)], buf, sem)

    def wait_l(buf, sem):
        pltpu.make_async_copy(mij_hbm.at[pl.ds(ebase, ch)], buf, sem).wait()

    start_l(0, mbuf, sm0)

    def pair(jj, carry):
        j0 = 2 * jj
        start_l(j0 + 1, mbuf1, sm1)
        wait_l(mbuf, sm0)
        pltpu.sync_copy(mbuf, macc.at[idx_v.at[j0]], add=True)
        lax.cond(j0 + 2 < nchw,
                 lambda: start_l(j0 + 2, mbuf, sm0),
                 lambda: None)
        wait_l(mbuf1, sm1)
        pltpu.sync_copy(mbuf1, macc.at[idx_v.at[j0 + 1]], add=True)
        return carry

    lax.fori_loop(0, nchw // 2, pair, 0)
    if nchw % 2:
        wait_l(mbuf, sm0)
        pltpu.sync_copy(mbuf, macc.at[idx_v.at[nchw - 1]], add=True)
    plsc.subcore_barrier()

    k = 0
    while k * ch < nps:
        rows = min(ch, nps - k * ch)
        base = s * nps + k * ch
        pltpu.sync_copy(macc.at[pl.ds(base, rows)],
                        pm_hbm.at[c].at[pl.ds(base, rows)])
        k += 1
    if tail:
        @pl.when(s == 0)
        def _():
            pltpu.sync_copy(macc.at[pl.ds(NS * nps, tail)],
                            pm_hbm.at[c].at[pl.ds(NS * nps, tail)])


def _sc_scatter(mij, srcm, n):
    e, d = mij.shape
    epw = e // NW
    dv = d // LANES
    mesh = plsc.VectorSubcoreMesh(core_axis_name="c", subcore_axis_name="s", num_cores=NC, num_subcores=NS)
    body = functools.partial(_sc_scatter_body, n, epw, CH, dv)
    k = pl.kernel(
        body,
        out_type=[
            jax.ShapeDtypeStruct((NC, n, d), jnp.float32),
        ],
        mesh=mesh,
        scratch_types=[
            pltpu.VMEM((epw // CH, CH), jnp.int32),
            pltpu.VMEM((CH, d), jnp.float32),
            pltpu.VMEM((CH, d), jnp.float32),
            pltpu.VMEM_SHARED((n, d), jnp.float32),
            pltpu.SemaphoreType.DMA,
            pltpu.SemaphoreType.DMA,
        ],
        compiler_params=pltpu.CompilerParams(needs_layout_passes=False),
    )
    return k(mij, srcm)[0]


def _sc_wd_body(n, epw, ch, dx,
                wd_hbm, src_hbm, pd_hbm,
                src_v, wbuf, wbuf1, dacc4, sw0, sw1):
    c = lax.axis_index("c")
    s = lax.axis_index("s")
    wid = s * NC + c
    ebase = wid * epw
    nch = epw // ch

    zvec = jnp.zeros((LANES,), jnp.float32)

    def zrow(i, carry):
        dacc4[pl.ds(i * LANES, LANES)] = zvec
        return carry

    lax.fori_loop(0, (4 * n) // LANES, zrow, 0)
    pltpu.sync_copy(src_hbm.at[pl.ds(ebase, epw)], src_v)

    il0 = lax.iota(jnp.int32, LANES)

    def start_l(j, buf, sem):
        pltpu.async_copy(wd_hbm.at[pl.ds(ebase + j * ch, ch)], buf, sem)

    def wait_l(buf, sem):
        pltpu.make_async_copy(wd_hbm.at[pl.ds(ebase, ch)], buf, sem).wait()

    def process(j, buf):
        off = j * ch
        for g in range(ch // LANES):
            il = il0 + g * LANES
            sv = src_v[pl.ds(off + g * LANES, LANES)] * 4
            for comp in range(4):
                cv = jnp.full((LANES,), comp, jnp.int32)
                val = plsc.load_gather(buf, [il, cv])
                plsc.addupdate_scatter(dacc4, [sv + cv], val)

    start_l(0, wbuf, sw0)

    def pair(jj, carry):
        j0 = 2 * jj
        start_l(j0 + 1, wbuf1, sw1)
        wait_l(wbuf, sw0)
        process(j0, wbuf)
        lax.cond(j0 + 2 < nch,
                 lambda: start_l(j0 + 2, wbuf, sw0),
                 lambda: None)
        wait_l(wbuf1, sw1)
        process(j0 + 1, wbuf1)
        return carry

    lax.fori_loop(0, nch // 2, pair, 0)
    if nch % 2:
        wait_l(wbuf, sw0)
        process(nch - 1, wbuf)
    pltpu.sync_copy(dacc4, pd_hbm.at[wid])


def _sc_wd(wd, src, n):
    e, dx = wd.shape
    epw = e // NW
    mesh = plsc.VectorSubcoreMesh(core_axis_name="c", subcore_axis_name="s",
                                  num_cores=NC, num_subcores=NS)
    body = functools.partial(_sc_wd_body, n, epw, CH, dx)
    k = pl.kernel(
        body,
        out_type=[
            jax.ShapeDtypeStruct((NW, 4 * n), jnp.float32),
        ],
        mesh=mesh,
        scratch_types=[
            pltpu.VMEM((epw,), jnp.int32),
            pltpu.VMEM((CH, dx), jnp.float32),
            pltpu.VMEM((CH, dx), jnp.float32),
            pltpu.VMEM((4 * n,), jnp.float32),
            pltpu.SemaphoreType.DMA,
            pltpu.SemaphoreType.DMA,
        ],
        compiler_params=pltpu.CompilerParams(needs_layout_passes=False),
    )
    return k(wd, src)[0]


def _psum_body(p_ref, o_ref):
    o_ref[...] = jnp.sum(p_ref[...], axis=0)


def _tc_psum(p):
    nw, m = p.shape
    return pl.pallas_call(
        _psum_body,
        grid=(1,),
        in_specs=[pl.BlockSpec((nw, m), lambda i: (0, 0))],
        out_specs=pl.BlockSpec((m,), lambda i: (0,)),
        out_shape=jax.ShapeDtypeStruct((m,), jnp.float32),
    )(p)


def _node_body(h_ref, xp_ref, pm_ref, pd_ref, wh1a_ref, wh1b_ref, bh1_ref,
               wh2_ref, bh2_ref, hout_ref, xout_ref):
    msum = pm_ref[0] + pm_ref[1]
    dsum = pd_ref[...]
    cnt = jnp.maximum(dsum[:, 3:4], 1.0)
    m_i = msum / cnt
    h = h_ref[...]
    t = _silu(jnp.dot(h, wh1a_ref[...], preferred_element_type=jnp.float32)
              + jnp.dot(m_i, wh1b_ref[...], preferred_element_type=jnp.float32)
              + bh1_ref[...])
    hout_ref[...] = h + jnp.dot(t, wh2_ref[...],
                                preferred_element_type=jnp.float32) + bh2_ref[...]
    pad = jnp.zeros((xp_ref.shape[0], xp_ref.shape[1] - 4), jnp.float32)
    xout_ref[...] = xp_ref[...] + jnp.concatenate([dsum / cnt, pad], axis=1)


def _tc_node(h, xp, pm, pd, wh1a, wh1b, bh1, wh2, bh2):
    n, d = h.shape
    dx = xp.shape[1]
    bn = _pick_block(n, (2000, 1000, 500, 400, 200, 100, 50, 40, 20, 10, 8))
    grid = (n // bn,)
    zero = lambda i: (0, 0)
    return pl.pallas_call(
        _node_body,
        grid=grid,
        in_specs=[
            pl.BlockSpec((bn, d), lambda i: (i, 0)),
            pl.BlockSpec((bn, dx), lambda i: (i, 0)),
            pl.BlockSpec((NC, bn, d), lambda i: (0, i, 0)),
            pl.BlockSpec((bn, 4), lambda i: (i, 0)),
            pl.BlockSpec((d, d), zero),
            pl.BlockSpec((d, d), zero),
            pl.BlockSpec((1, d), zero),
            pl.BlockSpec((d, d), zero),
            pl.BlockSpec((1, d), zero),
        ],
        out_specs=[
            pl.BlockSpec((bn, d), lambda i: (i, 0)),
            pl.BlockSpec((bn, dx), lambda i: (i, 0)),
        ],
        out_shape=[
            jax.ShapeDtypeStruct((n, d), jnp.float32),
            jax.ShapeDtypeStruct((n, dx), jnp.float32),
        ],
    )(h, xp, pm, pd, wh1a, wh1b, bh1, wh2, bh2)


def kernel(h, x, src, dst, distances, bpp_bias, msa_bias, chem_bias,
           relative_offset, chain_break_mask, W_m1, b_m1, W_m2, b_m2,
           W_h1, b_h1, W_h2, b_h2, W_c1, b_c1, W_c2, b_c2):
    n, d = h.shape
    e = src.shape[0]
    dx = 16

    xp = jnp.zeros((n, dx), jnp.float32).at[:, :3].set(x)
    xf = jnp.zeros((n, 4), jnp.float32).at[:, :3].set(x).reshape(-1)
    s8 = jnp.concatenate(
        [jnp.stack([distances, bpp_bias, msa_bias, chem_bias,
                    relative_offset, chain_break_mask], axis=-1),
         jnp.zeros((e, 2), jnp.float32)], axis=-1)
    w1a = W_m1[:d]
    w1b = W_m1[d:2 * d]
    w1s = jnp.concatenate([W_m1[2 * d:], jnp.zeros((8 - (W_m1.shape[0] - 2 * d), d),
                                                   jnp.float32)], axis=0)
    bm1 = b_m1.reshape(1, d)
    bm2 = b_m2.reshape(1, d)
    bc1 = b_c1.reshape(1, d)
    wc2t = W_c2.reshape(1, d)
    bc2 = b_c2.reshape(1, 1)
    wh1a = W_h1[:d]
    wh1b = W_h1[d:]
    bh1 = b_h1.reshape(1, d)
    bh2 = b_h2.reshape(1, d)
    srcm = src.reshape(NW, e // (NW * CH), CH)

    a, b = _tc_pre(h, w1a, w1b)
    m1pre, df = _sc_gather(a, b, xf, src, dst, dx)
    mij, wdw = _tc_edge(m1pre, df, s8, w1s, bm1,
                        W_m2.astype(jnp.bfloat16), bm2,
                        W_c1.astype(jnp.bfloat16), bc1, wc2t, bc2)
    pm = _sc_scatter(mij, srcm, n)
    pd4 = _tc_psum(_sc_wd(wdw, src, n)).reshape(n, 4)
    h_out, x_out_p = _tc_node(h, xp, pm, pd4, wh1a, wh1b, bh1, W_h2, bh2)
    return (h_out, x_out_p[:, :3])
